# baseline (speedup 1.0000x reference)
import functools

import jax
import jax.numpy as jnp
from jax import lax
from jax.experimental import pallas as pl
from jax.experimental.pallas import tpu as pltpu
from jax.experimental.pallas import tpu_sc as plsc

N_NODES = 1000000
EMBED_DIM = 64
BATCH = 16384

NUM_CORES = 2
NUM_SUBCORES = 16
LANES = 16
NUM_WORKERS = NUM_CORES * NUM_SUBCORES

ROWS_PER_WORKER = BATCH // NUM_WORKERS
BLK = 128
GROUP = 4
NUM_GROUPS = ROWS_PER_WORKER // GROUP


def _rsqrt_nr(s):
    i = plsc.bitcast(s, jnp.int32)
    i = jnp.int32(0x5F3759DF) - lax.shift_right_arithmetic(i, jnp.int32(1))
    y = plsc.bitcast(i, jnp.float32)
    for _ in range(3):
        y = y * (1.5 - 0.5 * s * y * y)
    return y


def _sc_body(nodes_hbm, tablet_hbm, outt_hbm, idx_v, blk_v, buft_v, tmp_v,
             *sems):
    wid = lax.axis_index("s") * NUM_CORES + lax.axis_index("c")
    base = wid * ROWS_PER_WORKER

    pltpu.sync_copy(nodes_hbm.at[pl.ds(base, ROWS_PER_WORKER)],
                    idx_v.at[pl.ds(0, ROWS_PER_WORKER)])
    idx_v[pl.ds(ROWS_PER_WORKER, LANES)] = jnp.zeros((LANES,), jnp.int32)

    def fire_group(g, half):
        v = idx_v[pl.ds(g * GROUP, LANES)]
        for t in range(GROUP):
            cw = lax.shift_right_logical(v[t], jnp.int32(7)) * jnp.int32(BLK)
            pltpu.async_copy(
                tablet_hbm.at[:, pl.ds(cw, BLK)],
                blk_v.at[half * GROUP + t],
                sems[half * GROUP + t])

    fire_group(0, 0)

    lanes = lax.iota(jnp.int32, LANES)
    quarter = lax.shift_right_logical(lanes, jnp.int32(2))
    sub = lax.bitwise_and(lanes, jnp.int32(3))

    def wave(g, carry):
        par = lax.bitwise_and(g, 1)

        for half in range(2):
            @pl.when(par == half)
            def _():
                for t in range(GROUP):
                    pltpu.make_async_copy(
                        tablet_hbm.at[:, pl.ds(0, BLK)],
                        blk_v.at[half * GROUP + t],
                        sems[half * GROUP + t]).wait()

        for half in range(2):
            @pl.when(jnp.logical_and(par == half, g < NUM_GROUPS - 1))
            def _():
                fire_group(g + 1, 1 - half)

        ids = plsc.load_gather(idx_v, [g * GROUP + quarter])
        q = lax.bitwise_and(ids, jnp.int32(BLK - 1))
        rvec = par * GROUP + quarter

        acc = jnp.zeros((LANES,), jnp.float32)
        for k in range(EMBED_DIM // 4):
            jv = jnp.full((LANES,), 4 * k, jnp.int32) + sub
            x = plsc.load_gather(blk_v, [rvec, jv, q])
            acc = acc + x * x
        tmp_v[...] = acc
        acc = acc + plsc.load_gather(tmp_v, [lax.bitwise_xor(lanes,
                                                             jnp.int32(1))])
        tmp_v[...] = acc
        acc = acc + plsc.load_gather(tmp_v, [lax.bitwise_xor(lanes,
                                                             jnp.int32(2))])
        r = _rsqrt_nr(jnp.maximum(acc, jnp.float32(1e-24)))

        cols = g * GROUP + quarter
        for k in range(EMBED_DIM // 4):
            jv = jnp.full((LANES,), 4 * k, jnp.int32) + sub
            x = plsc.load_gather(blk_v, [rvec, jv, q])
            plsc.store_scatter(buft_v, [jv, cols], x * r)

        return carry

    lax.fori_loop(0, NUM_GROUPS, wave, 0)

    pltpu.sync_copy(buft_v, outt_hbm.at[:, pl.ds(base, ROWS_PER_WORKER)])


@jax.jit
def _pale_embedding_sc(nodes, table):
    mesh = plsc.VectorSubcoreMesh(core_axis_name="c", subcore_axis_name="s")
    outt = pl.kernel(
        _sc_body,
        out_type=jax.ShapeDtypeStruct((EMBED_DIM, BATCH), jnp.float32),
        mesh=mesh,
        scratch_types=[
            pltpu.VMEM((ROWS_PER_WORKER + LANES,), jnp.int32),
            pltpu.VMEM((2 * GROUP, EMBED_DIM, BLK), jnp.float32),
            pltpu.VMEM((EMBED_DIM, ROWS_PER_WORKER), jnp.float32),
            pltpu.VMEM((LANES,), jnp.float32),
        ] + [pltpu.SemaphoreType.DMA] * (2 * GROUP),
        compiler_params=pltpu.CompilerParams(needs_layout_passes=False),
    )(nodes, table.T)
    return outt.T


def kernel(nodes, table):
    return _pale_embedding_sc(nodes, table)
